# baseline (speedup 1.0000x reference)
import jax
import jax.numpy as jnp
from jax import lax
from jax.experimental import pallas as pl

N = 10000
T = 12
G = 2
NTG = T // G
CM = 64
GC = G * CM
CIN = 32
EH = 320000
EV = 160000


def _k2_body(x_ref, cnth_ref, cntv_ref, wh_ref, wv_ref,
             tabh_ref, tabv_ref, dish_ref, disv_ref):
    dh = lax.rsqrt(cnth_ref[...] + 1.0)
    dv = lax.rsqrt(cntv_ref[...] + 1.0)
    dims = (((0,), (0,)), ((), ()))
    hs, vs = [], []
    for g in range(G):
        xg = x_ref[g]
        hs.append(lax.dot_general(xg * dh, wh_ref[0:CIN, :], dims,
                                  preferred_element_type=jnp.float32))
        vs.append(lax.dot_general(xg * dv, wv_ref[0:CIN, :], dims,
                                  preferred_element_type=jnp.float32))
    tabh_ref[0] = jnp.concatenate(hs, axis=1)
    tabv_ref[0] = jnp.concatenate(vs, axis=1)
    dish_ref[...] = dh
    disv_ref[...] = dv


def _k2(x_t, cnth, cntv, W_h, W_v):
    return pl.pallas_call(
        _k2_body,
        grid=(NTG,),
        in_specs=[
            pl.BlockSpec((G, CIN, N), lambda tg: (tg, 0, 0)),
            pl.BlockSpec((1, N), lambda tg: (0, 0)),
            pl.BlockSpec((1, N), lambda tg: (0, 0)),
            pl.BlockSpec((CM, CM), lambda tg: (0, 0)),
            pl.BlockSpec((CM, CM), lambda tg: (0, 0)),
        ],
        out_specs=[
            pl.BlockSpec((1, N, GC), lambda tg: (tg, 0, 0)),
            pl.BlockSpec((1, N, GC), lambda tg: (tg, 0, 0)),
            pl.BlockSpec((1, N), lambda tg: (0, 0)),
            pl.BlockSpec((1, N), lambda tg: (0, 0)),
        ],
        out_shape=[
            jax.ShapeDtypeStruct((NTG, N, GC), jnp.float32),
            jax.ShapeDtypeStruct((NTG, N, GC), jnp.float32),
            jax.ShapeDtypeStruct((1, N), jnp.float32),
            jax.ShapeDtypeStruct((1, N), jnp.float32),
        ],
    )(x_t, cnth, cntv, W_h, W_v)


NB = 1000


def _k4_body(oh_ref, ov_ref, tabh_ref, tabv_ref, dish_ref, disv_ref, b_ref,
             g_ref, be_ref, y_ref):
    dish = dish_ref[...]
    disv = disv_ref[...]
    for g in range(G):
        sl = pl.ds(g * CM, CM)
        oh = oh_ref[0, :, sl]
        ov = ov_ref[0, :, sl]
        hh = tabh_ref[0, :, sl]
        hv = tabv_ref[0, :, sl]
        z = dish * (oh + hh) + disv * (ov + hv) + b_ref[...]
        z = jnp.maximum(z, 0.0)
        mu = jnp.mean(z, axis=1, keepdims=True)
        zc = z - mu
        var = jnp.mean(zc * zc, axis=1, keepdims=True)
        y_ref[g] = zc * lax.rsqrt(var + 1e-5) * g_ref[...] + be_ref[...]


def _k4(oh, ov, tabh, tabv, dish_c, disv_c, bsum, gamma2, beta2):
    return pl.pallas_call(
        _k4_body,
        grid=(NTG, N // NB),
        in_specs=[
            pl.BlockSpec((1, NB, GC), lambda tg, nb: (tg, nb, 0)),
            pl.BlockSpec((1, NB, GC), lambda tg, nb: (tg, nb, 0)),
            pl.BlockSpec((1, NB, GC), lambda tg, nb: (tg, nb, 0)),
            pl.BlockSpec((1, NB, GC), lambda tg, nb: (tg, nb, 0)),
            pl.BlockSpec((NB, 1), lambda tg, nb: (nb, 0)),
            pl.BlockSpec((NB, 1), lambda tg, nb: (nb, 0)),
            pl.BlockSpec((1, CM), lambda tg, nb: (0, 0)),
            pl.BlockSpec((1, CM), lambda tg, nb: (0, 0)),
            pl.BlockSpec((1, CM), lambda tg, nb: (0, 0)),
        ],
        out_specs=pl.BlockSpec((G, NB, CM), lambda tg, nb: (tg, nb, 0)),
        out_shape=jax.ShapeDtypeStruct((T, N, CM), jnp.float32),
    )(oh, ov, tabh, tabv, dish_c, disv_c, bsum, gamma2, beta2)


def kernel(x_room, edge_index_h, edge_index_v, W_h, b_h, W_v, b_v, gamma, beta):
    x_t = jnp.transpose(x_room[0], (1, 0, 2))
    rh, ch = edge_index_h[0], edge_index_h[1]
    rv, cv = edge_index_v[0], edge_index_v[1]

    cnth = jax.ops.segment_sum(jnp.ones(EH, jnp.float32), ch, num_segments=N)
    cntv = jax.ops.segment_sum(jnp.ones(EV, jnp.float32), cv, num_segments=N)
    tabh, tabv, dish, disv = _k2(x_t, cnth[None, :], cntv[None, :], W_h, W_v)

    def seg(tab, rows, cols):
        return jax.vmap(lambda tt: jax.ops.segment_sum(tt[rows], cols,
                                                       num_segments=N))(tab)
    oh = seg(tabh, rh, ch)
    ov = seg(tabv, rv, cv)
    y = _k4(oh, ov, tabh, tabv,
            dish.reshape(N, 1), disv.reshape(N, 1),
            (b_h + b_v)[None, :], gamma[None, :], beta[None, :])
    return jnp.transpose(y, (2, 0, 1))[None]


# speedup vs baseline: 6.8552x; 6.8552x over previous
import jax
import jax.numpy as jnp
from jax import lax
from jax.experimental import pallas as pl

N = 10000
T = 12
G = 2
NTG = T // G
CM = 64
GC = G * CM
CIN = 32
EH = 320000
EV = 160000


def _k2_body(x_ref, cnth_ref, cntv_ref, wh_ref, wv_ref,
             tabh_ref, tabv_ref, dish_ref, disv_ref):
    dh = lax.rsqrt(cnth_ref[...] + 1.0)
    dv = lax.rsqrt(cntv_ref[...] + 1.0)
    dims = (((0,), (0,)), ((), ()))
    hs, vs = [], []
    for g in range(G):
        xg = x_ref[g]
        hs.append(lax.dot_general(xg * dh, wh_ref[0:CIN, :], dims,
                                  preferred_element_type=jnp.float32))
        vs.append(lax.dot_general(xg * dv, wv_ref[0:CIN, :], dims,
                                  preferred_element_type=jnp.float32))
    tabh_ref[0] = jnp.concatenate(hs, axis=1)
    tabv_ref[0] = jnp.concatenate(vs, axis=1)
    dish_ref[...] = dh
    disv_ref[...] = dv


def _k2(x_t, cnth, cntv, W_h, W_v):
    return pl.pallas_call(
        _k2_body,
        grid=(NTG,),
        in_specs=[
            pl.BlockSpec((G, CIN, N), lambda tg: (tg, 0, 0)),
            pl.BlockSpec((1, N), lambda tg: (0, 0)),
            pl.BlockSpec((1, N), lambda tg: (0, 0)),
            pl.BlockSpec((CM, CM), lambda tg: (0, 0)),
            pl.BlockSpec((CM, CM), lambda tg: (0, 0)),
        ],
        out_specs=[
            pl.BlockSpec((1, N, GC), lambda tg: (tg, 0, 0)),
            pl.BlockSpec((1, N, GC), lambda tg: (tg, 0, 0)),
            pl.BlockSpec((1, N), lambda tg: (0, 0)),
            pl.BlockSpec((1, N), lambda tg: (0, 0)),
        ],
        out_shape=[
            jax.ShapeDtypeStruct((NTG, N, GC), jnp.float32),
            jax.ShapeDtypeStruct((NTG, N, GC), jnp.float32),
            jax.ShapeDtypeStruct((1, N), jnp.float32),
            jax.ShapeDtypeStruct((1, N), jnp.float32),
        ],
    )(x_t, cnth, cntv, W_h, W_v)


NB = 1000


def _k4_body(oh_ref, ov_ref, tabh_ref, tabv_ref, dish_ref, disv_ref, b_ref,
             g_ref, be_ref, y_ref):
    dish = dish_ref[...]
    disv = disv_ref[...]
    for g in range(G):
        sl = pl.ds(g * CM, CM)
        oh = oh_ref[0, :, sl]
        ov = ov_ref[0, :, sl]
        hh = tabh_ref[0, :, sl]
        hv = tabv_ref[0, :, sl]
        z = dish * (oh + hh) + disv * (ov + hv) + b_ref[...]
        z = jnp.maximum(z, 0.0)
        mu = jnp.mean(z, axis=1, keepdims=True)
        zc = z - mu
        var = jnp.mean(zc * zc, axis=1, keepdims=True)
        y_ref[g] = zc * lax.rsqrt(var + 1e-5) * g_ref[...] + be_ref[...]


def _k4(oh, ov, tabh, tabv, dish_c, disv_c, bsum, gamma2, beta2):
    return pl.pallas_call(
        _k4_body,
        grid=(NTG, N // NB),
        in_specs=[
            pl.BlockSpec((1, NB, GC), lambda tg, nb: (tg, nb, 0)),
            pl.BlockSpec((1, NB, GC), lambda tg, nb: (tg, nb, 0)),
            pl.BlockSpec((1, NB, GC), lambda tg, nb: (tg, nb, 0)),
            pl.BlockSpec((1, NB, GC), lambda tg, nb: (tg, nb, 0)),
            pl.BlockSpec((NB, 1), lambda tg, nb: (nb, 0)),
            pl.BlockSpec((NB, 1), lambda tg, nb: (nb, 0)),
            pl.BlockSpec((1, CM), lambda tg, nb: (0, 0)),
            pl.BlockSpec((1, CM), lambda tg, nb: (0, 0)),
            pl.BlockSpec((1, CM), lambda tg, nb: (0, 0)),
        ],
        out_specs=pl.BlockSpec((G, NB, CM), lambda tg, nb: (tg, nb, 0)),
        out_shape=jax.ShapeDtypeStruct((T, N, CM), jnp.float32),
    )(oh, ov, tabh, tabv, dish_c, disv_c, bsum, gamma2, beta2)


def kernel(x_room, edge_index_h, edge_index_v, W_h, b_h, W_v, b_v, gamma, beta):
    x_t = jnp.transpose(x_room[0], (1, 0, 2))
    rh, ch = edge_index_h[0], edge_index_h[1]
    rv, cv = edge_index_v[0], edge_index_v[1]

    cnth = jax.ops.segment_sum(jnp.ones(EH, jnp.float32), ch, num_segments=N)
    cntv = jax.ops.segment_sum(jnp.ones(EV, jnp.float32), cv, num_segments=N)
    tabh, tabv, dish, disv = _k2(x_t, cnth[None, :], cntv[None, :], W_h, W_v)

    def seg(tab, rows, cols):
        return jnp.stack([jax.ops.segment_sum(tab[g][rows], cols,
                                              num_segments=N)
                          for g in range(NTG)])
    oh = seg(tabh, rh, ch)
    ov = seg(tabv, rv, cv)
    y = _k4(oh, ov, tabh, tabv,
            dish.reshape(N, 1), disv.reshape(N, 1),
            (b_h + b_v)[None, :], gamma[None, :], beta[None, :])
    return jnp.transpose(y, (2, 0, 1))[None]
